# baseline (speedup 1.0000x reference)
import functools

import jax
import jax.numpy as jnp
from jax import lax
from jax.experimental import pallas as pl
from jax.experimental.pallas import tpu as pltpu
from jax.experimental.pallas import tpu_sc as plsc

_D = 768
_E = 64
_K = 8
_T = 32768
_BLK = 4096
_NW = 32
_RPT = _T // _NW
_GRP = _RPT // 16


def _enc(v, lane):
    b = lax.bitcast_convert_type(v, jnp.int32)
    b = b ^ (lax.shift_right_arithmetic(b, 31) & jnp.int32(0x7FFFFFFF))
    return (b & jnp.int32(~63)) | (jnp.int32(63) - lane)


def _gate_body(x_ref, wt_ref, enc_ref):
    x = x_ref[...]
    wt = wt_ref[...]
    logits = jnp.dot(x, wt, preferred_element_type=jnp.float32)
    lane = lax.broadcasted_iota(jnp.int32, logits.shape, 1)
    enc_ref[...] = _enc(logits, lane)


def _tc_gate(inp, wt):
    return pl.pallas_call(
        _gate_body,
        grid=(_T // _BLK,),
        in_specs=[
            pl.BlockSpec((_BLK, _D), lambda i: (i, 0)),
            pl.BlockSpec((_D, _E), lambda i: (0, 0)),
        ],
        out_specs=pl.BlockSpec((_BLK, _E), lambda i: (i, 0)),
        out_shape=jax.ShapeDtypeStruct((_T, _E), jnp.int32),
        compiler_params=pltpu.CompilerParams(
            dimension_semantics=("arbitrary",),
        ),
    )(inp, wt)


def _sc_topk(enc_flat):
    mesh = plsc.VectorSubcoreMesh(core_axis_name="c", subcore_axis_name="s")

    @functools.partial(
        pl.kernel,
        mesh=mesh,
        out_type=[
            jax.ShapeDtypeStruct((_T * _K,), jnp.int32),
            jax.ShapeDtypeStruct((_T * _K,), jnp.float32),
        ],
        scratch_types=[
            pltpu.VMEM((_RPT * _E,), jnp.int32),
            pltpu.VMEM((_RPT * _K,), jnp.int32),
            pltpu.VMEM((_RPT * _K,), jnp.float32),
        ],
        compiler_params=pltpu.CompilerParams(needs_layout_passes=False),
    )
    def sc(enc_hbm, idx_hbm, scr_hbm, enc_v, idx_v, scr_v):
        wid = lax.axis_index("s") * 2 + lax.axis_index("c")
        base = wid * _RPT
        pltpu.sync_copy(enc_hbm.at[pl.ds(base * _E, _RPT * _E)], enc_v)
        lanes = lax.iota(jnp.int32, 16)
        neg = jnp.full((16,), -(2**31), jnp.int32)

        def group(g, carry):
            fb = g * (16 * _E) + lanes * _E
            ob = g * (16 * _K) + lanes * _K
            vals = []
            for k in range(_K):
                blocks = []
                for j8 in range(0, _E, 8):
                    es = [plsc.load_gather(enc_v, [fb + (j8 + t)])
                          for t in range(8)]
                    while len(es) > 1:
                        es = [jnp.maximum(a, b)
                              for a, b in zip(es[::2], es[1::2])]
                    blocks.append(es[0])
                while len(blocks) > 1:
                    blocks = [jnp.maximum(a, b)
                              for a, b in zip(blocks[::2], blocks[1::2])]
                cur = blocks[0]
                idx_k = jnp.int32(63) - (cur & jnp.int32(63))
                plsc.store_scatter(enc_v, [fb + idx_k], neg)
                b = cur ^ (lax.shift_right_arithmetic(cur, 31)
                           & jnp.int32(0x7FFFFFFF))
                vals.append(plsc.bitcast(b, jnp.float32))
                plsc.store_scatter(idx_v, [ob + k], idx_k)
            es = [jnp.exp(v - vals[0]) for v in vals]
            tot = es[0]
            for e in es[1:]:
                tot = tot + e
            for k in range(_K):
                plsc.store_scatter(scr_v, [ob + k], es[k] / tot)
            return carry

        lax.fori_loop(0, _GRP, group, 0)
        pltpu.sync_copy(idx_v, idx_hbm.at[pl.ds(base * _K, _RPT * _K)])
        pltpu.sync_copy(scr_v, scr_hbm.at[pl.ds(base * _K, _RPT * _K)])

    return sc(enc_flat)


def kernel(inp, W):
    enc = _tc_gate(inp, W.T)
    idx_f, scr_f = _sc_topk(enc.reshape(_T * _E))
    return (idx_f.reshape(_T, _K), scr_f.reshape(_T, _K))


# speedup vs baseline: 4.4916x; 4.4916x over previous
import functools

import jax
import jax.numpy as jnp
from jax import lax
from jax.experimental import pallas as pl
from jax.experimental.pallas import tpu as pltpu
from jax.experimental.pallas import tpu_sc as plsc

_D = 768
_E = 64
_K = 8
_T = 32768
_BLK = 4096
_NW = 32
_RPT = _T // _NW
_GRP = _RPT // 16


def _enc(v, lane):
    b = lax.bitcast_convert_type(v, jnp.int32)
    b = b ^ (lax.shift_right_arithmetic(b, 31) & jnp.int32(0x7FFFFFFF))
    return (b & jnp.int32(~63)) | (jnp.int32(63) - lane)


def _dec(m):
    b = m ^ (lax.shift_right_arithmetic(m, 31) & jnp.int32(0x7FFFFFFF))
    return lax.bitcast_convert_type(b, jnp.float32)


def _gate_body(x_ref, wt_ref, enc_ref):
    x = x_ref[...]
    wt = wt_ref[...]
    logits = jnp.dot(x, wt, preferred_element_type=jnp.float32)
    lt = logits.T
    lane = lax.broadcasted_iota(jnp.int32, lt.shape, 0)
    enc_ref[...] = _enc(lt, lane)


def _tc_gate(inp, wt):
    return pl.pallas_call(
        _gate_body,
        grid=(_T // _BLK,),
        in_specs=[
            pl.BlockSpec((_BLK, _D), lambda i: (i, 0)),
            pl.BlockSpec((_D, _E), lambda i: (0, 0)),
        ],
        out_specs=pl.BlockSpec((_E, _BLK), lambda i: (0, i)),
        out_shape=jax.ShapeDtypeStruct((_E, _T), jnp.int32),
        compiler_params=pltpu.CompilerParams(
            dimension_semantics=("arbitrary",),
        ),
    )(inp, wt)


def _sc_topk(enc_t):
    mesh = plsc.VectorSubcoreMesh(core_axis_name="c", subcore_axis_name="s")

    @functools.partial(
        pl.kernel,
        mesh=mesh,
        out_type=[
            jax.ShapeDtypeStruct((_K, _T), jnp.int32),
            jax.ShapeDtypeStruct((_K, _T), jnp.float32),
        ],
        scratch_types=[
            pltpu.VMEM((_E, _RPT), jnp.int32),
            pltpu.VMEM((_K, _RPT), jnp.int32),
            pltpu.VMEM((_K, _RPT), jnp.float32),
        ],
        compiler_params=pltpu.CompilerParams(needs_layout_passes=False),
    )
    def sc(enc_hbm, idx_hbm, scr_hbm, enc_v, idx_v, scr_v):
        wid = lax.axis_index("s") * 2 + lax.axis_index("c")
        base = wid * _RPT
        pltpu.sync_copy(enc_hbm.at[:, pl.ds(base, _RPT)], enc_v)
        neg = jnp.full((16,), -(2**31), jnp.int32)

        def group(g, carry):
            cols = pl.ds(g * 16, 16)
            t = [neg] * _K
            for j in range(_E):
                c = enc_v[j, cols]
                for i in range(_K):
                    hi = jnp.maximum(t[i], c)
                    c = jnp.minimum(t[i], c)
                    t[i] = hi
            vals = [_dec(tk) for tk in t]
            es = [jnp.exp(v - vals[0]) for v in vals]
            tot = es[0]
            for e in es[1:]:
                tot = tot + e
            for k in range(_K):
                idx_v[k, cols] = jnp.int32(63) - (t[k] & jnp.int32(63))
                scr_v[k, cols] = es[k] / tot
            return carry

        lax.fori_loop(0, _GRP, group, 0)
        pltpu.sync_copy(idx_v, idx_hbm.at[:, pl.ds(base, _RPT)])
        pltpu.sync_copy(scr_v, scr_hbm.at[:, pl.ds(base, _RPT)])

    return sc(enc_t)


def kernel(inp, W):
    enc_t = _tc_gate(inp, W.T)
    idx_t, scr_t = _sc_topk(enc_t)
    return (idx_t.T, scr_t.T)


# speedup vs baseline: 8.1163x; 1.8070x over previous
import jax
import jax.numpy as jnp
from jax import lax
from jax.experimental import pallas as pl
from jax.experimental.pallas import tpu as pltpu

_D = 768
_E = 64
_K = 8
_T = 32768
_H = _T // 2
_BLK = 2048
_NB = _H // _BLK


def _enc(v, lane):
    b = lax.bitcast_convert_type(v, jnp.int32)
    b = b ^ (lax.shift_right_arithmetic(b, 31) & jnp.int32(0x7FFFFFFF))
    return (b & jnp.int32(~63)) | (jnp.int32(63) - lane)


def _dec(m):
    b = m ^ (lax.shift_right_arithmetic(m, 31) & jnp.int32(0x7FFFFFFF))
    return lax.bitcast_convert_type(b, jnp.float32)


def _body(xa_ref, xb_ref, wt_ref, ia_ref, ib_ref, sa_ref, sb_ref):
    x = jnp.concatenate([xa_ref[...], xb_ref[...]], axis=0)
    wt = wt_ref[...]
    logits = jnp.dot(x, wt, preferred_element_type=jnp.float32)
    lt = logits.T
    lane = lax.broadcasted_iota(jnp.int32, lt.shape, 0)
    enc = _enc(lt, lane)
    ms = []
    for k in range(_K):
        m = jnp.max(enc, axis=0, keepdims=True)
        ms.append(m)
        if k + 1 < _K:
            enc = jnp.where(enc == m, jnp.int32(-(2**31)), enc)
    mk = jnp.concatenate(ms, axis=0)
    idx = jnp.int32(63) - (mk & jnp.int32(63))
    vals = _dec(mk)
    e = jnp.exp(vals - vals[0:1])
    scr = e / jnp.sum(e, axis=0, keepdims=True)
    ia_ref[...] = idx[:, :_BLK]
    ib_ref[...] = idx[:, _BLK:]
    sa_ref[...] = scr[:, :_BLK]
    sb_ref[...] = scr[:, _BLK:]


def _router(inp, wt):
    return pl.pallas_call(
        _body,
        grid=(_NB,),
        in_specs=[
            pl.BlockSpec((_BLK, _D), lambda i: (i, 0)),
            pl.BlockSpec((_BLK, _D), lambda i: (i + _NB, 0)),
            pl.BlockSpec((_D, _E), lambda i: (0, 0)),
        ],
        out_specs=[
            pl.BlockSpec((_K, _BLK), lambda i: (0, i)),
            pl.BlockSpec((_K, _BLK), lambda i: (0, i)),
            pl.BlockSpec((_K, _BLK), lambda i: (0, i)),
            pl.BlockSpec((_K, _BLK), lambda i: (0, i)),
        ],
        out_shape=[
            jax.ShapeDtypeStruct((_K, _H), jnp.int32),
            jax.ShapeDtypeStruct((_K, _H), jnp.int32),
            jax.ShapeDtypeStruct((_K, _H), jnp.float32),
            jax.ShapeDtypeStruct((_K, _H), jnp.float32),
        ],
        compiler_params=pltpu.CompilerParams(
            dimension_semantics=("arbitrary",),
        ),
    )(inp, inp, wt)


def kernel(inp, W):
    ia, ib, sa, sb = _router(inp, W.T)
    idx = jnp.concatenate([ia, ib], axis=1).T
    scr = jnp.concatenate([sa, sb], axis=1).T
    return (idx, scr)
